# baseline (speedup 1.0000x reference)
import functools

import jax
import jax.numpy as jnp
import numpy as np
from jax import lax
from jax.experimental import pallas as pl
from jax.experimental.pallas import tpu as pltpu
from jax.experimental.pallas import tpu_sc as plsc

THRESH_V = float(-np.log(np.float32(0.7)))
N_MIN_V = 262144

_N = 4194304
_ROWS = 4096
_COLS = 1024
_BLK_ROWS = 256
_GRID = _ROWS // _BLK_ROWS

_NW = 32
_PER_W = _N // _NW
_CHUNK = 32768
_NCHUNK = _PER_W // _CHUNK
_UNROLL = 4

_SC_MESH = plsc.VectorSubcoreMesh(core_axis_name="c", subcore_axis_name="s",
                                  num_cores=2, num_subcores=16)


def _loss_stats_kernel(p_ref, t_ref, loss_ref, stats_ref):
    p = p_ref[...]
    t = t_ref[...]
    log_p = jnp.maximum(jnp.log(p), -100.0)
    log_1mp = jnp.maximum(jnp.log(1.0 - p), -100.0)
    loss = -(t * log_p + (1.0 - t) * log_1mp)
    loss_ref[...] = loss
    m = loss > THRESH_V
    s = jnp.sum(jnp.where(m, loss, 0.0))
    c = jnp.sum(m.astype(jnp.float32))

    @pl.when(pl.program_id(0) == 0)
    def _init():
        stats_ref[0, 0] = s
        stats_ref[0, 1] = c

    @pl.when(pl.program_id(0) != 0)
    def _acc():
        stats_ref[0, 0] += s
        stats_ref[0, 1] += c


def _loss_and_stats(p2d, t2d):
    return pl.pallas_call(
        _loss_stats_kernel,
        grid=(_GRID,),
        in_specs=[
            pl.BlockSpec((_BLK_ROWS, _COLS), lambda i: (i, 0)),
            pl.BlockSpec((_BLK_ROWS, _COLS), lambda i: (i, 0)),
        ],
        out_specs=[
            pl.BlockSpec((_BLK_ROWS, _COLS), lambda i: (i, 0)),
            pl.BlockSpec((1, 2), lambda i: (0, 0), memory_space=pltpu.SMEM),
        ],
        out_shape=[
            jax.ShapeDtypeStruct((_ROWS, _COLS), jnp.float32),
            jax.ShapeDtypeStruct((1, 2), jnp.float32),
        ],
    )(p2d, t2d)


def _make_hist_kernel(shift, nb, match_shift):
    @functools.partial(
        pl.kernel,
        out_type=jax.ShapeDtypeStruct((_NW, nb), jnp.float32),
        mesh=_SC_MESH,
        compiler_params=pltpu.CompilerParams(needs_layout_passes=False),
        scratch_types=[
            pltpu.VMEM((_CHUNK,), jnp.int32),
            pltpu.VMEM((_CHUNK,), jnp.int32),
            pltpu.VMEM((16,), jnp.int32),
            pltpu.VMEM((16 * nb,), jnp.float32),
            pltpu.VMEM((nb,), jnp.float32),
            pltpu.SemaphoreType.DMA,
            pltpu.SemaphoreType.DMA,
        ],
    )
    def hist_kernel(bits_hbm, pref_hbm, cnt_out,
                    chunk0_v, chunk1_v, pref_v, cnt_v, rcnt_v, sem0, sem1):
        wid = lax.axis_index("s") * 2 + lax.axis_index("c")
        base = wid * _PER_W
        bufs = (chunk0_v, chunk1_v)
        sems = (sem0, sem1)

        pltpu.async_copy(bits_hbm.at[pl.ds(base, _CHUNK)], bufs[0], sems[0])

        zeros16 = jnp.zeros((16,), jnp.float32)

        def zbody(i, carry):
            for j in range(8):
                cnt_v[pl.ds(i * 128 + j * 16, 16)] = zeros16
            return carry

        lax.fori_loop(0, nb // 8, zbody, 0)

        pltpu.sync_copy(pref_hbm, pref_v)
        pref = pref_v[...]
        lane_nb = lax.iota(jnp.int32, 16) * nb
        ones16 = jnp.ones((16,), jnp.float32)

        for c in range(_NCHUNK):
            cur = bufs[c % 2]
            pltpu.make_async_copy(
                bits_hbm.at[pl.ds(base + c * _CHUNK, _CHUNK)],
                cur, sems[c % 2]).wait()
            if c + 1 < _NCHUNK:
                pltpu.async_copy(
                    bits_hbm.at[pl.ds(base + (c + 1) * _CHUNK, _CHUNK)],
                    bufs[(c + 1) % 2], sems[(c + 1) % 2])

            @plsc.parallel_loop(0, _CHUNK // 16, 1, unroll=_UNROLL)
            def _chunk_body(i):
                bits = cur[pl.ds(i * 16, 16)]
                b = jnp.right_shift(bits, shift) & (nb - 1)
                idx = lane_nb + b
                if match_shift is None:
                    plsc.addupdate_scatter(cnt_v, [idx], ones16)
                else:
                    match = jnp.right_shift(bits, match_shift) == pref
                    plsc.addupdate_scatter(cnt_v, [idx], ones16, mask=match)

        def rbody(j, carry):
            ca = cnt_v[pl.ds(j * 16, 16)]
            for l in range(1, 16):
                ca = ca + cnt_v[pl.ds(l * nb + j * 16, 16)]
            rcnt_v[pl.ds(j * 16, 16)] = ca
            return carry

        lax.fori_loop(0, nb // 16, rbody, 0)

        pltpu.sync_copy(rcnt_v, cnt_out.at[wid])

    return hist_kernel


_hist_r0 = _make_hist_kernel(21, 2048, None)
_hist_r1 = _make_hist_kernel(10, 2048, 21)


_NB_F = 1024


@functools.partial(
    pl.kernel,
    out_type=(
        jax.ShapeDtypeStruct((_NW, _NB_F), jnp.float32),
        jax.ShapeDtypeStruct((_NW, 16), jnp.float32),
    ),
    mesh=_SC_MESH,
    compiler_params=pltpu.CompilerParams(needs_layout_passes=False),
    scratch_types=[
        pltpu.VMEM((_CHUNK,), jnp.float32),
        pltpu.VMEM((_CHUNK,), jnp.float32),
        pltpu.VMEM((64,), jnp.float32),
        pltpu.VMEM((16 * _NB_F,), jnp.float32),
        pltpu.VMEM((_NB_F,), jnp.float32),
        pltpu.VMEM((16,), jnp.float32),
        pltpu.SemaphoreType.DMA,
        pltpu.SemaphoreType.DMA,
    ],
)
def _final_kernel(loss_hbm, params_hbm, cnt_out, sum_out,
                  chunk0_v, chunk1_v, params_v, cnt_v, rcnt_v, s_v,
                  sem0, sem1):
    wid = lax.axis_index("s") * 2 + lax.axis_index("c")
    base = wid * _PER_W
    bufs = (chunk0_v, chunk1_v)
    sems = (sem0, sem1)

    pltpu.async_copy(loss_hbm.at[pl.ds(base, _CHUNK)], bufs[0], sems[0])
    pltpu.sync_copy(params_hbm, params_v)
    lo = params_v[pl.ds(0, 16)]
    hi = params_v[pl.ds(16, 16)]
    sc_a = params_v[pl.ds(32, 16)]
    sc_b = params_v[pl.ds(48, 16)]

    zeros16 = jnp.zeros((16,), jnp.float32)
    ones16 = jnp.ones((16,), jnp.float32)
    lane_nb = lax.iota(jnp.int32, 16) * _NB_F
    max_bin = jnp.full((16,), _NB_F - 1, jnp.int32)
    zero_i = jnp.zeros((16,), jnp.int32)

    def zbody(i, carry):
        for j in range(8):
            cnt_v[pl.ds(i * 128 + j * 16, 16)] = zeros16
        return carry

    lax.fori_loop(0, 16 * _NB_F // 128, zbody, 0)

    accs = tuple(zeros16 for _ in range(_UNROLL))
    for c in range(_NCHUNK):
        cur = bufs[c % 2]
        pltpu.make_async_copy(
            loss_hbm.at[pl.ds(base + c * _CHUNK, _CHUNK)],
            cur, sems[c % 2]).wait()
        if c + 1 < _NCHUNK:
            pltpu.async_copy(
                loss_hbm.at[pl.ds(base + (c + 1) * _CHUNK, _CHUNK)],
                bufs[(c + 1) % 2], sems[(c + 1) % 2])

        @plsc.parallel_loop(0, _CHUNK // 16, 1, unroll=_UNROLL)
        def _scatter_body(i):
            x = cur[pl.ds(i * 16, 16)]
            m_in = (x >= lo) & (x < hi)
            u = ((x - lo) * sc_a) * sc_b
            b = jnp.minimum(jnp.maximum(u.astype(jnp.int32), zero_i),
                            max_bin)
            plsc.addupdate_scatter(cnt_v, [lane_nb + b], ones16, mask=m_in)

        def sbody(i, carry):
            out = list(carry)
            for j in range(_UNROLL):
                x = cur[pl.ds(i * (16 * _UNROLL) + j * 16, 16)]
                out[j] = out[j] + jnp.where(x >= hi, x, zeros16)
            return tuple(out)

        accs = lax.fori_loop(0, _CHUNK // (16 * _UNROLL), sbody, accs)

    s_acc = accs[0]
    for j in range(1, _UNROLL):
        s_acc = s_acc + accs[j]

    def rbody(j, carry):
        ca = cnt_v[pl.ds(j * 16, 16)]
        for l in range(1, 16):
            ca = ca + cnt_v[pl.ds(l * _NB_F + j * 16, 16)]
        rcnt_v[pl.ds(j * 16, 16)] = ca
        return carry

    lax.fori_loop(0, _NB_F // 16, rbody, 0)

    s_v[...] = s_acc
    pltpu.sync_copy(rcnt_v, cnt_out.at[wid])
    pltpu.sync_copy(s_v, sum_out.at[wid])


def _pick_bin(cnt_w, nb, k_rem):
    cnt = jnp.sum(cnt_w, axis=0)
    cnt_d = cnt[::-1]
    csum = jnp.cumsum(cnt_d)
    lt = csum < k_rem
    i_star = jnp.minimum(jnp.sum(lt.astype(jnp.int32)), nb - 1)
    b_star = (nb - 1 - i_star).astype(jnp.int32)
    above_cnt = jnp.max(jnp.where(lt, csum, 0.0))
    return b_star, k_rem - above_cnt


def kernel(output, target):
    p2d = output.reshape(_ROWS, _COLS)
    t2d = target.reshape(_ROWS, _COLS)
    loss, stats = _loss_and_stats(p2d, t2d)
    masked_sum = stats[0, 0]
    count = stats[0, 1]

    loss_flat = loss.reshape(_N)
    bits_flat = lax.bitcast_convert_type(loss_flat, jnp.int32)
    k = jnp.float32(N_MIN_V)

    pref0 = jnp.zeros((16,), jnp.int32)
    cnt0 = _hist_r0(bits_flat, pref0)
    b0, k_rem = _pick_bin(cnt0, 2048, k)

    pref1 = jnp.broadcast_to(b0, (16,))
    cnt1 = _hist_r1(bits_flat, pref1)
    b1, k_rem = _pick_bin(cnt1, 2048, k_rem)

    prefix22 = b0 * 2048 + b1
    e22 = prefix22 >> 13
    lo_bits = prefix22 << 10
    hi_bits = (prefix22 + 1) << 10
    lo = lax.bitcast_convert_type(lo_bits, jnp.float32)
    hi = lax.bitcast_convert_type(hi_bits, jnp.float32)
    sc_a = jnp.float32(2.0) ** 75
    sc_b = jnp.ldexp(jnp.float32(1.0), 75 - jnp.maximum(e22, 1))
    params = jnp.concatenate([
        jnp.broadcast_to(lo, (16,)),
        jnp.broadcast_to(hi, (16,)),
        jnp.broadcast_to(sc_a, (16,)),
        jnp.broadcast_to(sc_b, (16,)),
    ])

    cnt_w, sum_w = _final_kernel(loss_flat, params)
    cnt2 = jnp.sum(cnt_w, axis=0)
    b2, k_rem = _pick_bin(cnt_w, _NB_F, k_rem)
    vals = lax.bitcast_convert_type(
        lo_bits + jnp.arange(_NB_F, dtype=jnp.int32), jnp.float32)
    ws = jnp.sum(jnp.where(jnp.arange(_NB_F) > b2, cnt2 * vals, 0.0))
    pivot = lax.bitcast_convert_type(lo_bits + b2, jnp.float32)
    s_above = jnp.sum(sum_w)
    mean_top = (s_above + ws + k_rem * pivot) / k

    mean_masked = masked_sum / jnp.maximum(count, 1.0)
    return jnp.where(count > k, mean_masked, mean_top)


# speedup vs baseline: 2.6570x; 2.6570x over previous
import functools

import jax
import jax.numpy as jnp
import numpy as np
from jax import lax
from jax.experimental import pallas as pl
from jax.experimental.pallas import tpu as pltpu
from jax.experimental.pallas import tpu_sc as plsc

THRESH_V = float(-np.log(np.float32(0.7)))
N_MIN_V = 262144

_N = 4194304
_ROWS = 4096
_COLS = 1024
_BLK_ROWS = 512
_GRID = _ROWS // _BLK_ROWS

_NW = 32
_PER_W = _N // _NW
_CHUNK = 32768
_NCHUNK = _PER_W // _CHUNK
_UNROLL = 4

_SC_MESH = plsc.VectorSubcoreMesh(core_axis_name="c", subcore_axis_name="s",
                                  num_cores=2, num_subcores=16)


def _loss_stats_kernel(p_ref, t_ref, loss_ref, stats_ref):
    p = p_ref[...]
    t = t_ref[...]
    log_p = jnp.maximum(jnp.log(p), -100.0)
    log_1mp = jnp.maximum(jnp.log(1.0 - p), -100.0)
    loss = -(t * log_p + (1.0 - t) * log_1mp)
    loss_ref[...] = loss
    m = loss > THRESH_V
    s = jnp.sum(jnp.where(m, loss, 0.0))
    c = jnp.sum(m.astype(jnp.float32))

    @pl.when(pl.program_id(0) == 0)
    def _init():
        stats_ref[0, 0] = s
        stats_ref[0, 1] = c

    @pl.when(pl.program_id(0) != 0)
    def _acc():
        stats_ref[0, 0] += s
        stats_ref[0, 1] += c


def _loss_and_stats(p2d, t2d):
    return pl.pallas_call(
        _loss_stats_kernel,
        grid=(_GRID,),
        in_specs=[
            pl.BlockSpec((_BLK_ROWS, _COLS), lambda i: (i, 0)),
            pl.BlockSpec((_BLK_ROWS, _COLS), lambda i: (i, 0)),
        ],
        out_specs=[
            pl.BlockSpec((_BLK_ROWS, _COLS), lambda i: (i, 0)),
            pl.BlockSpec((1, 2), lambda i: (0, 0), memory_space=pltpu.SMEM),
        ],
        out_shape=[
            jax.ShapeDtypeStruct((_ROWS, _COLS), jnp.float32),
            jax.ShapeDtypeStruct((1, 2), jnp.float32),
        ],
    )(p2d, t2d)


def _make_hist_kernel(shift, nb, match_shift):
    @functools.partial(
        pl.kernel,
        out_type=jax.ShapeDtypeStruct((_NW, nb), jnp.float32),
        mesh=_SC_MESH,
        compiler_params=pltpu.CompilerParams(needs_layout_passes=False),
        scratch_types=[
            pltpu.VMEM((_CHUNK,), jnp.int32),
            pltpu.VMEM((_CHUNK,), jnp.int32),
            pltpu.VMEM((16,), jnp.int32),
            pltpu.VMEM((16 * nb,), jnp.float32),
            pltpu.VMEM((nb,), jnp.float32),
            pltpu.SemaphoreType.DMA,
            pltpu.SemaphoreType.DMA,
        ],
    )
    def hist_kernel(bits_hbm, pref_hbm, cnt_out,
                    chunk0_v, chunk1_v, pref_v, cnt_v, rcnt_v, sem0, sem1):
        wid = lax.axis_index("s") * 2 + lax.axis_index("c")
        base = wid * _PER_W
        bufs = (chunk0_v, chunk1_v)
        sems = (sem0, sem1)

        pltpu.async_copy(bits_hbm.at[pl.ds(base, _CHUNK)], bufs[0], sems[0])

        zeros16 = jnp.zeros((16,), jnp.float32)

        def zbody(i, carry):
            for j in range(8):
                cnt_v[pl.ds(i * 128 + j * 16, 16)] = zeros16
            return carry

        lax.fori_loop(0, nb // 8, zbody, 0)

        pltpu.sync_copy(pref_hbm, pref_v)
        pref = pref_v[...]
        lane_nb = lax.iota(jnp.int32, 16) * nb
        ones16 = jnp.ones((16,), jnp.float32)

        for c in range(_NCHUNK):
            cur = bufs[c % 2]
            pltpu.make_async_copy(
                bits_hbm.at[pl.ds(base + c * _CHUNK, _CHUNK)],
                cur, sems[c % 2]).wait()
            if c + 1 < _NCHUNK:
                pltpu.async_copy(
                    bits_hbm.at[pl.ds(base + (c + 1) * _CHUNK, _CHUNK)],
                    bufs[(c + 1) % 2], sems[(c + 1) % 2])

            @plsc.parallel_loop(0, _CHUNK // 16, 1, unroll=_UNROLL)
            def _chunk_body(i):
                bits = cur[pl.ds(i * 16, 16)]
                b = jnp.right_shift(bits, shift) & (nb - 1)
                idx = lane_nb + b
                if match_shift is None:
                    plsc.addupdate_scatter(cnt_v, [idx], ones16)
                else:
                    match = jnp.right_shift(bits, match_shift) == pref
                    plsc.addupdate_scatter(cnt_v, [idx], ones16, mask=match)

        def rbody(j, carry):
            ca = cnt_v[pl.ds(j * 16, 16)]
            for l in range(1, 16):
                ca = ca + cnt_v[pl.ds(l * nb + j * 16, 16)]
            rcnt_v[pl.ds(j * 16, 16)] = ca
            return carry

        lax.fori_loop(0, nb // 16, rbody, 0)

        pltpu.sync_copy(rcnt_v, cnt_out.at[wid])

    return hist_kernel


_hist_r0 = _make_hist_kernel(21, 2048, None)
_hist_r1 = _make_hist_kernel(10, 2048, 21)


_NB_F = 1024


@functools.partial(
    pl.kernel,
    out_type=(
        jax.ShapeDtypeStruct((_NW, _NB_F), jnp.float32),
        jax.ShapeDtypeStruct((_NW, 16), jnp.float32),
    ),
    mesh=_SC_MESH,
    compiler_params=pltpu.CompilerParams(needs_layout_passes=False),
    scratch_types=[
        pltpu.VMEM((_CHUNK,), jnp.float32),
        pltpu.VMEM((_CHUNK,), jnp.float32),
        pltpu.VMEM((64,), jnp.float32),
        pltpu.VMEM((16 * _NB_F,), jnp.float32),
        pltpu.VMEM((_NB_F,), jnp.float32),
        pltpu.VMEM((16,), jnp.float32),
        pltpu.SemaphoreType.DMA,
        pltpu.SemaphoreType.DMA,
    ],
)
def _final_kernel(loss_hbm, params_hbm, cnt_out, sum_out,
                  chunk0_v, chunk1_v, params_v, cnt_v, rcnt_v, s_v,
                  sem0, sem1):
    wid = lax.axis_index("s") * 2 + lax.axis_index("c")
    base = wid * _PER_W
    bufs = (chunk0_v, chunk1_v)
    sems = (sem0, sem1)

    pltpu.async_copy(loss_hbm.at[pl.ds(base, _CHUNK)], bufs[0], sems[0])
    pltpu.sync_copy(params_hbm, params_v)
    lo = params_v[pl.ds(0, 16)]
    hi = params_v[pl.ds(16, 16)]
    sc_a = params_v[pl.ds(32, 16)]
    sc_b = params_v[pl.ds(48, 16)]

    zeros16 = jnp.zeros((16,), jnp.float32)
    ones16 = jnp.ones((16,), jnp.float32)
    lane_nb = lax.iota(jnp.int32, 16) * _NB_F
    max_bin = jnp.full((16,), _NB_F - 1, jnp.int32)
    zero_i = jnp.zeros((16,), jnp.int32)

    def zbody(i, carry):
        for j in range(8):
            cnt_v[pl.ds(i * 128 + j * 16, 16)] = zeros16
        return carry

    lax.fori_loop(0, 16 * _NB_F // 128, zbody, 0)

    accs = tuple(zeros16 for _ in range(_UNROLL))
    for c in range(_NCHUNK):
        cur = bufs[c % 2]
        pltpu.make_async_copy(
            loss_hbm.at[pl.ds(base + c * _CHUNK, _CHUNK)],
            cur, sems[c % 2]).wait()
        if c + 1 < _NCHUNK:
            pltpu.async_copy(
                loss_hbm.at[pl.ds(base + (c + 1) * _CHUNK, _CHUNK)],
                bufs[(c + 1) % 2], sems[(c + 1) % 2])

        @plsc.parallel_loop(0, _CHUNK // 16, 1, unroll=_UNROLL)
        def _scatter_body(i):
            x = cur[pl.ds(i * 16, 16)]
            m_in = (x >= lo) & (x < hi)
            u = ((x - lo) * sc_a) * sc_b
            b = jnp.minimum(jnp.maximum(u.astype(jnp.int32), zero_i),
                            max_bin)
            plsc.addupdate_scatter(cnt_v, [lane_nb + b], ones16, mask=m_in)

        def sbody(i, carry):
            out = list(carry)
            for j in range(_UNROLL):
                x = cur[pl.ds(i * (16 * _UNROLL) + j * 16, 16)]
                out[j] = out[j] + jnp.where(x >= hi, x, zeros16)
            return tuple(out)

        accs = lax.fori_loop(0, _CHUNK // (16 * _UNROLL), sbody, accs)

    s_acc = accs[0]
    for j in range(1, _UNROLL):
        s_acc = s_acc + accs[j]

    def rbody(j, carry):
        ca = cnt_v[pl.ds(j * 16, 16)]
        for l in range(1, 16):
            ca = ca + cnt_v[pl.ds(l * _NB_F + j * 16, 16)]
        rcnt_v[pl.ds(j * 16, 16)] = ca
        return carry

    lax.fori_loop(0, _NB_F // 16, rbody, 0)

    s_v[...] = s_acc
    pltpu.sync_copy(rcnt_v, cnt_out.at[wid])
    pltpu.sync_copy(s_v, sum_out.at[wid])


def _pick_bin(cnt_w, nb, k_rem):
    cnt = jnp.sum(cnt_w, axis=0)
    cnt_d = cnt[::-1]
    csum = jnp.cumsum(cnt_d)
    lt = csum < k_rem
    i_star = jnp.minimum(jnp.sum(lt.astype(jnp.int32)), nb - 1)
    b_star = (nb - 1 - i_star).astype(jnp.int32)
    above_cnt = jnp.max(jnp.where(lt, csum, 0.0))
    return b_star, k_rem - above_cnt


def _mean_top_k(loss):
    loss_flat = loss.reshape(_N)
    bits_flat = lax.bitcast_convert_type(loss_flat, jnp.int32)
    k = jnp.float32(N_MIN_V)

    pref0 = jnp.zeros((16,), jnp.int32)
    cnt0 = _hist_r0(bits_flat, pref0)
    b0, k_rem = _pick_bin(cnt0, 2048, k)

    pref1 = jnp.broadcast_to(b0, (16,))
    cnt1 = _hist_r1(bits_flat, pref1)
    b1, k_rem = _pick_bin(cnt1, 2048, k_rem)

    prefix22 = b0 * 2048 + b1
    e22 = prefix22 >> 13
    lo_bits = prefix22 << 10
    hi_bits = (prefix22 + 1) << 10
    lo = lax.bitcast_convert_type(lo_bits, jnp.float32)
    hi = lax.bitcast_convert_type(hi_bits, jnp.float32)
    sc_a = jnp.float32(2.0) ** 75
    sc_b = jnp.ldexp(jnp.float32(1.0), 75 - jnp.maximum(e22, 1))
    params = jnp.concatenate([
        jnp.broadcast_to(lo, (16,)),
        jnp.broadcast_to(hi, (16,)),
        jnp.broadcast_to(sc_a, (16,)),
        jnp.broadcast_to(sc_b, (16,)),
    ])

    cnt_w, sum_w = _final_kernel(loss_flat, params)
    cnt2 = jnp.sum(cnt_w, axis=0)
    b2, k_rem = _pick_bin(cnt_w, _NB_F, k_rem)
    vals = lax.bitcast_convert_type(
        lo_bits + jnp.arange(_NB_F, dtype=jnp.int32), jnp.float32)
    ws = jnp.sum(jnp.where(jnp.arange(_NB_F) > b2, cnt2 * vals, 0.0))
    pivot = lax.bitcast_convert_type(lo_bits + b2, jnp.float32)
    s_above = jnp.sum(sum_w)
    return (s_above + ws + k_rem * pivot) / k


def kernel(output, target):
    p2d = output.reshape(_ROWS, _COLS)
    t2d = target.reshape(_ROWS, _COLS)
    loss, stats = _loss_and_stats(p2d, t2d)
    masked_sum = stats[0, 0]
    count = stats[0, 1]
    mean_masked = masked_sum / jnp.maximum(count, 1.0)
    return lax.cond(count > jnp.float32(N_MIN_V),
                    lambda _: mean_masked,
                    _mean_top_k,
                    loss)


# speedup vs baseline: 2.7961x; 1.0523x over previous
import functools

import jax
import jax.numpy as jnp
import numpy as np
from jax import lax
from jax.experimental import pallas as pl
from jax.experimental.pallas import tpu as pltpu
from jax.experimental.pallas import tpu_sc as plsc

THRESH_V = float(-np.log(np.float32(0.7)))
N_MIN_V = 262144

_N = 4194304
_ROWS = 4096
_COLS = 1024
_BLK_ROWS = 512
_GRID = _ROWS // _BLK_ROWS

_NW = 32
_PER_W = _N // _NW
_CHUNK = 32768
_NCHUNK = _PER_W // _CHUNK
_UNROLL = 4

_SC_MESH = plsc.VectorSubcoreMesh(core_axis_name="c", subcore_axis_name="s",
                                  num_cores=2, num_subcores=16)


def _bce(p, t):
    log_p = jnp.maximum(jnp.log(p), -100.0)
    log_1mp = jnp.maximum(jnp.log(1.0 - p), -100.0)
    return -(t * log_p + (1.0 - t) * log_1mp)


def _stats_kernel(p_ref, t_ref, stats_ref):
    loss = _bce(p_ref[...], t_ref[...])
    m = loss > THRESH_V
    s = jnp.sum(jnp.where(m, loss, 0.0))
    c = jnp.sum(m.astype(jnp.float32))

    @pl.when(pl.program_id(0) == 0)
    def _init():
        stats_ref[0, 0] = s
        stats_ref[0, 1] = c

    @pl.when(pl.program_id(0) != 0)
    def _acc():
        stats_ref[0, 0] += s
        stats_ref[0, 1] += c


def _masked_stats(p2d, t2d):
    return pl.pallas_call(
        _stats_kernel,
        grid=(_GRID,),
        in_specs=[
            pl.BlockSpec((_BLK_ROWS, _COLS), lambda i: (i, 0)),
            pl.BlockSpec((_BLK_ROWS, _COLS), lambda i: (i, 0)),
        ],
        out_specs=pl.BlockSpec((1, 2), lambda i: (0, 0),
                               memory_space=pltpu.SMEM),
        out_shape=jax.ShapeDtypeStruct((1, 2), jnp.float32),
    )(p2d, t2d)


def _loss_kernel(p_ref, t_ref, loss_ref):
    loss_ref[...] = _bce(p_ref[...], t_ref[...])


def _loss_array(p2d, t2d):
    return pl.pallas_call(
        _loss_kernel,
        grid=(_GRID,),
        in_specs=[
            pl.BlockSpec((_BLK_ROWS, _COLS), lambda i: (i, 0)),
            pl.BlockSpec((_BLK_ROWS, _COLS), lambda i: (i, 0)),
        ],
        out_specs=pl.BlockSpec((_BLK_ROWS, _COLS), lambda i: (i, 0)),
        out_shape=jax.ShapeDtypeStruct((_ROWS, _COLS), jnp.float32),
    )(p2d, t2d)


def _make_hist_kernel(shift, nb, match_shift):
    @functools.partial(
        pl.kernel,
        out_type=jax.ShapeDtypeStruct((_NW, nb), jnp.float32),
        mesh=_SC_MESH,
        compiler_params=pltpu.CompilerParams(needs_layout_passes=False),
        scratch_types=[
            pltpu.VMEM((_CHUNK,), jnp.int32),
            pltpu.VMEM((_CHUNK,), jnp.int32),
            pltpu.VMEM((16,), jnp.int32),
            pltpu.VMEM((16 * nb,), jnp.float32),
            pltpu.VMEM((nb,), jnp.float32),
            pltpu.SemaphoreType.DMA,
            pltpu.SemaphoreType.DMA,
        ],
    )
    def hist_kernel(bits_hbm, pref_hbm, cnt_out,
                    chunk0_v, chunk1_v, pref_v, cnt_v, rcnt_v, sem0, sem1):
        wid = lax.axis_index("s") * 2 + lax.axis_index("c")
        base = wid * _PER_W
        bufs = (chunk0_v, chunk1_v)
        sems = (sem0, sem1)

        pltpu.async_copy(bits_hbm.at[pl.ds(base, _CHUNK)], bufs[0], sems[0])

        zeros16 = jnp.zeros((16,), jnp.float32)

        def zbody(i, carry):
            for j in range(8):
                cnt_v[pl.ds(i * 128 + j * 16, 16)] = zeros16
            return carry

        lax.fori_loop(0, nb // 8, zbody, 0)

        pltpu.sync_copy(pref_hbm, pref_v)
        pref = pref_v[...]
        lane_nb = lax.iota(jnp.int32, 16) * nb
        ones16 = jnp.ones((16,), jnp.float32)

        for c in range(_NCHUNK):
            cur = bufs[c % 2]
            pltpu.make_async_copy(
                bits_hbm.at[pl.ds(base + c * _CHUNK, _CHUNK)],
                cur, sems[c % 2]).wait()
            if c + 1 < _NCHUNK:
                pltpu.async_copy(
                    bits_hbm.at[pl.ds(base + (c + 1) * _CHUNK, _CHUNK)],
                    bufs[(c + 1) % 2], sems[(c + 1) % 2])

            @plsc.parallel_loop(0, _CHUNK // 16, 1, unroll=_UNROLL)
            def _chunk_body(i):
                bits = cur[pl.ds(i * 16, 16)]
                b = jnp.right_shift(bits, shift) & (nb - 1)
                idx = lane_nb + b
                if match_shift is None:
                    plsc.addupdate_scatter(cnt_v, [idx], ones16)
                else:
                    match = jnp.right_shift(bits, match_shift) == pref
                    plsc.addupdate_scatter(cnt_v, [idx], ones16, mask=match)

        def rbody(j, carry):
            ca = cnt_v[pl.ds(j * 16, 16)]
            for l in range(1, 16):
                ca = ca + cnt_v[pl.ds(l * nb + j * 16, 16)]
            rcnt_v[pl.ds(j * 16, 16)] = ca
            return carry

        lax.fori_loop(0, nb // 16, rbody, 0)

        pltpu.sync_copy(rcnt_v, cnt_out.at[wid])

    return hist_kernel


_hist_r0 = _make_hist_kernel(21, 2048, None)
_hist_r1 = _make_hist_kernel(10, 2048, 21)


_NB_F = 1024


@functools.partial(
    pl.kernel,
    out_type=(
        jax.ShapeDtypeStruct((_NW, _NB_F), jnp.float32),
        jax.ShapeDtypeStruct((_NW, 16), jnp.float32),
    ),
    mesh=_SC_MESH,
    compiler_params=pltpu.CompilerParams(needs_layout_passes=False),
    scratch_types=[
        pltpu.VMEM((_CHUNK,), jnp.float32),
        pltpu.VMEM((_CHUNK,), jnp.float32),
        pltpu.VMEM((64,), jnp.float32),
        pltpu.VMEM((16 * _NB_F,), jnp.float32),
        pltpu.VMEM((_NB_F,), jnp.float32),
        pltpu.VMEM((16,), jnp.float32),
        pltpu.SemaphoreType.DMA,
        pltpu.SemaphoreType.DMA,
    ],
)
def _final_kernel(loss_hbm, params_hbm, cnt_out, sum_out,
                  chunk0_v, chunk1_v, params_v, cnt_v, rcnt_v, s_v,
                  sem0, sem1):
    wid = lax.axis_index("s") * 2 + lax.axis_index("c")
    base = wid * _PER_W
    bufs = (chunk0_v, chunk1_v)
    sems = (sem0, sem1)

    pltpu.async_copy(loss_hbm.at[pl.ds(base, _CHUNK)], bufs[0], sems[0])
    pltpu.sync_copy(params_hbm, params_v)
    lo = params_v[pl.ds(0, 16)]
    hi = params_v[pl.ds(16, 16)]
    sc_a = params_v[pl.ds(32, 16)]
    sc_b = params_v[pl.ds(48, 16)]

    zeros16 = jnp.zeros((16,), jnp.float32)
    ones16 = jnp.ones((16,), jnp.float32)
    lane_nb = lax.iota(jnp.int32, 16) * _NB_F
    max_bin = jnp.full((16,), _NB_F - 1, jnp.int32)
    zero_i = jnp.zeros((16,), jnp.int32)

    def zbody(i, carry):
        for j in range(8):
            cnt_v[pl.ds(i * 128 + j * 16, 16)] = zeros16
        return carry

    lax.fori_loop(0, 16 * _NB_F // 128, zbody, 0)

    accs = tuple(zeros16 for _ in range(_UNROLL))
    for c in range(_NCHUNK):
        cur = bufs[c % 2]
        pltpu.make_async_copy(
            loss_hbm.at[pl.ds(base + c * _CHUNK, _CHUNK)],
            cur, sems[c % 2]).wait()
        if c + 1 < _NCHUNK:
            pltpu.async_copy(
                loss_hbm.at[pl.ds(base + (c + 1) * _CHUNK, _CHUNK)],
                bufs[(c + 1) % 2], sems[(c + 1) % 2])

        @plsc.parallel_loop(0, _CHUNK // 16, 1, unroll=_UNROLL)
        def _scatter_body(i):
            x = cur[pl.ds(i * 16, 16)]
            m_in = (x >= lo) & (x < hi)
            u = ((x - lo) * sc_a) * sc_b
            b = jnp.minimum(jnp.maximum(u.astype(jnp.int32), zero_i),
                            max_bin)
            plsc.addupdate_scatter(cnt_v, [lane_nb + b], ones16, mask=m_in)

        def sbody(i, carry):
            out = list(carry)
            for j in range(_UNROLL):
                x = cur[pl.ds(i * (16 * _UNROLL) + j * 16, 16)]
                out[j] = out[j] + jnp.where(x >= hi, x, zeros16)
            return tuple(out)

        accs = lax.fori_loop(0, _CHUNK // (16 * _UNROLL), sbody, accs)

    s_acc = accs[0]
    for j in range(1, _UNROLL):
        s_acc = s_acc + accs[j]

    def rbody(j, carry):
        ca = cnt_v[pl.ds(j * 16, 16)]
        for l in range(1, 16):
            ca = ca + cnt_v[pl.ds(l * _NB_F + j * 16, 16)]
        rcnt_v[pl.ds(j * 16, 16)] = ca
        return carry

    lax.fori_loop(0, _NB_F // 16, rbody, 0)

    s_v[...] = s_acc
    pltpu.sync_copy(rcnt_v, cnt_out.at[wid])
    pltpu.sync_copy(s_v, sum_out.at[wid])


def _pick_bin(cnt_w, nb, k_rem):
    cnt = jnp.sum(cnt_w, axis=0)
    cnt_d = cnt[::-1]
    csum = jnp.cumsum(cnt_d)
    lt = csum < k_rem
    i_star = jnp.minimum(jnp.sum(lt.astype(jnp.int32)), nb - 1)
    b_star = (nb - 1 - i_star).astype(jnp.int32)
    above_cnt = jnp.max(jnp.where(lt, csum, 0.0))
    return b_star, k_rem - above_cnt


def _mean_top_k(pt):
    p2d, t2d = pt
    loss_flat = _loss_array(p2d, t2d).reshape(_N)
    bits_flat = lax.bitcast_convert_type(loss_flat, jnp.int32)
    k = jnp.float32(N_MIN_V)

    pref0 = jnp.zeros((16,), jnp.int32)
    cnt0 = _hist_r0(bits_flat, pref0)
    b0, k_rem = _pick_bin(cnt0, 2048, k)

    pref1 = jnp.broadcast_to(b0, (16,))
    cnt1 = _hist_r1(bits_flat, pref1)
    b1, k_rem = _pick_bin(cnt1, 2048, k_rem)

    prefix22 = b0 * 2048 + b1
    e22 = prefix22 >> 13
    lo_bits = prefix22 << 10
    hi_bits = (prefix22 + 1) << 10
    lo = lax.bitcast_convert_type(lo_bits, jnp.float32)
    hi = lax.bitcast_convert_type(hi_bits, jnp.float32)
    sc_a = jnp.float32(2.0) ** 75
    sc_b = jnp.ldexp(jnp.float32(1.0), 75 - jnp.maximum(e22, 1))
    params = jnp.concatenate([
        jnp.broadcast_to(lo, (16,)),
        jnp.broadcast_to(hi, (16,)),
        jnp.broadcast_to(sc_a, (16,)),
        jnp.broadcast_to(sc_b, (16,)),
    ])

    cnt_w, sum_w = _final_kernel(loss_flat, params)
    cnt2 = jnp.sum(cnt_w, axis=0)
    b2, k_rem = _pick_bin(cnt_w, _NB_F, k_rem)
    vals = lax.bitcast_convert_type(
        lo_bits + jnp.arange(_NB_F, dtype=jnp.int32), jnp.float32)
    ws = jnp.sum(jnp.where(jnp.arange(_NB_F) > b2, cnt2 * vals, 0.0))
    pivot = lax.bitcast_convert_type(lo_bits + b2, jnp.float32)
    s_above = jnp.sum(sum_w)
    return (s_above + ws + k_rem * pivot) / k


def kernel(output, target):
    p2d = output.reshape(_ROWS, _COLS)
    t2d = target.reshape(_ROWS, _COLS)
    stats = _masked_stats(p2d, t2d)
    masked_sum = stats[0, 0]
    count = stats[0, 1]
    mean_masked = masked_sum / jnp.maximum(count, 1.0)
    return lax.cond(count > jnp.float32(N_MIN_V),
                    lambda _: mean_masked,
                    _mean_top_k,
                    (p2d, t2d))
